# baseline (speedup 1.0000x reference)
import functools

import jax
import jax.numpy as jnp
from jax.experimental import pallas as pl

DQ = 128
DM = 512


def _tc_full(idx_ref, ct_ref, p_ref, sw_ref, sb_ref, w_ref, b_ref,
             o_ref, *, h, w):
    nc = ct_ref.shape[0]
    wc = w_ref[0:DQ, :]
    wp = w_ref[DQ:2 * DQ, :]
    ws = w_ref[2 * DQ:3 * DQ, :]
    size_e = float(h) * sw_ref[0:1, :] + float(w) * sw_ref[1:2, :] + sb_ref[0:1, :]
    const = jnp.dot(size_e, ws, preferred_element_type=jnp.float32) + b_ref[0:1, :]
    zt = jnp.dot(ct_ref[...], wc, preferred_element_type=jnp.float32)
    zt = jnp.concatenate(
        [zt, jnp.zeros((DQ - nc - 1, DM), jnp.float32), const], axis=0)
    lanes = jax.lax.broadcasted_iota(jnp.int32, (h, w, DQ), 2)
    oh = ((lanes == idx_ref[...][:, :, None]) | (lanes == DQ - 1)
          ).astype(jnp.float32)
    dn = (((2,), (0,)), ((), ()))
    lhs = jnp.concatenate([oh, p_ref[...]], axis=2)
    rhs = jnp.concatenate([zt, wp], axis=0)
    o_ref[...] = jax.lax.dot_general(lhs, rhs, dn,
                                     preferred_element_type=jnp.float32)


def kernel(grid, color_table, pos_emb, size_W, size_b, combine_W, combine_b):
    h, w = grid.shape
    return pl.pallas_call(
        functools.partial(_tc_full, h=h, w=w),
        out_shape=jax.ShapeDtypeStruct((h, w, DM), jnp.float32),
    )(
        grid.astype(jnp.int32),
        color_table,
        pos_emb[:h, :w],
        size_W,
        size_b.reshape(1, DQ),
        combine_W,
        combine_b.reshape(1, DM),
    )
